# Initial kernel scaffold; baseline (speedup 1.0000x reference)
import jax
import jax.numpy as jnp
from jax.experimental import pallas as pl


def kernel(real_nodes, batch, vn_init, router_w1, router_b1, router_g, router_beta, router_w2, router_b2, r2v_w, r2v_b, r2v_g, r2v_beta, attn_in_w, attn_in_b, attn_out_w, attn_out_b, v2v_g, v2v_beta, v2r_w, v2r_b, v2r_g, v2r_beta, gru_w_ih, gru_w_hh, gru_b_ih, gru_b_hh):
    raise NotImplementedError("write your pallas kernel here")


# speedup vs baseline: 8.2264x; 8.2264x over previous
import functools

import jax
import jax.numpy as jnp
from jax import lax
from jax.experimental import pallas as pl

N = 50000
H = 512
V = 4
K = 2
G = 64
NH = 4
HD = H // NH
GV = G * V
BLK = 2000


def _ln(x, g, b, eps=1e-5):
    m = jnp.mean(x, axis=-1, keepdims=True)
    v = jnp.mean((x - m) ** 2, axis=-1, keepdims=True)
    return (x - m) * lax.rsqrt(v + eps) * g + b


def _dot_t(a, b):
    return lax.dot_general(a, b, (((1,), (1,)), ((), ())),
                           preferred_element_type=jnp.float32)


def _routing_matrix(batch, wts):
    c = lax.broadcasted_iota(jnp.int32, (1, GV), 1)
    g_of_c = c // V
    v_of_c = c % V
    b1h = (batch == g_of_c).astype(jnp.float32)
    wv = jnp.zeros_like(b1h)
    for v in range(V):
        wv = wv + wts[:, v:v + 1] * (v_of_c == v).astype(jnp.float32)
    return b1h * wv


def _phase_a(x_ref, b_ref, w1_ref, b1_ref, g1_ref, be1_ref, w2_ref, b2_ref,
             wr_ref, br_ref, gr_ref, ber_ref, agg_ref, wts_ref):
    i = pl.program_id(0)
    x = x_ref[...]
    h = jnp.maximum(_ln(_dot_t(x, w1_ref[...]) + b1_ref[...],
                        g1_ref[...], be1_ref[...]), 0.0)
    logits = _dot_t(h, w2_ref[...]) + b2_ref[...]
    idx4 = lax.broadcasted_iota(jnp.int32, logits.shape, 1)
    m1 = jnp.max(logits, axis=1, keepdims=True)
    i1 = jnp.min(jnp.where(logits == m1, idx4, V), axis=1, keepdims=True)
    oh1 = idx4 == i1
    l2 = jnp.where(oh1, -jnp.inf, logits)
    m2 = jnp.max(l2, axis=1, keepdims=True)
    i2 = jnp.min(jnp.where(l2 == m2, idx4, V), axis=1, keepdims=True)
    wts = (oh1 | (idx4 == i2)).astype(jnp.float32)
    wts_ref[...] = wts
    msg = jnp.maximum(_ln(_dot_t(x, wr_ref[...]) + br_ref[...],
                          gr_ref[...], ber_ref[...]), 0.0)
    s = _routing_matrix(b_ref[...], wts)

    @pl.when(i == 0)
    def _():
        agg_ref[...] = jnp.zeros_like(agg_ref)

    agg_ref[...] += lax.dot_general(s, msg, (((0,), (0,)), ((), ())),
                                    preferred_element_type=jnp.float32)


def _phase_b(agg_ref, vnt_ref, wi_ref, bi_ref, wo_ref, bo_ref,
             g2_ref, be2_ref, wv_ref, bv_ref, gv_ref, bev_ref, m2r_ref):
    vn = agg_ref[...] + vnt_ref[...]
    qkv = _dot_t(vn, wi_ref[...]) + bi_ref[...]
    r = lax.broadcasted_iota(jnp.int32, (GV, GV), 0)
    c = lax.broadcasted_iota(jnp.int32, (GV, GV), 1)
    mask = (r // V) == (c // V)
    scale = 1.0 / (HD ** 0.5)
    outs = []
    for hh in range(NH):
        q = qkv[:, hh * HD:(hh + 1) * HD]
        k = qkv[:, H + hh * HD:H + (hh + 1) * HD]
        v = qkv[:, 2 * H + hh * HD:2 * H + (hh + 1) * HD]
        s = _dot_t(q, k) * scale
        s = jnp.where(mask, s, -1e30)
        s = s - jnp.max(s, axis=1, keepdims=True)
        e = jnp.exp(s)
        p = e / jnp.sum(e, axis=1, keepdims=True)
        outs.append(jnp.dot(p, v, preferred_element_type=jnp.float32))
    o = jnp.concatenate(outs, axis=1)
    o = _dot_t(o, wo_ref[...]) + bo_ref[...]
    vn2 = _ln(vn + o, g2_ref[...], be2_ref[...])
    m2r_ref[...] = jnp.maximum(_ln(_dot_t(vn2, wv_ref[...]) + bv_ref[...],
                                   gv_ref[...], bev_ref[...]), 0.0)


def _phase_c(x_ref, b_ref, wts_ref, m2r_ref, wih_ref, whh_ref,
             bih_ref, bhh_ref, out_ref):
    x = x_ref[...]
    s = _routing_matrix(b_ref[...], wts_ref[...])
    rm = jnp.dot(s, m2r_ref[...], preferred_element_type=jnp.float32)
    gi = _dot_t(rm, wih_ref[...]) + bih_ref[...]
    gh = _dot_t(x, whh_ref[...]) + bhh_ref[...]
    r = jax.nn.sigmoid(gi[:, :H] + gh[:, :H])
    z = jax.nn.sigmoid(gi[:, H:2 * H] + gh[:, H:2 * H])
    n = jnp.tanh(gi[:, 2 * H:] + r * gh[:, 2 * H:])
    out_ref[...] = (1.0 - z) * n + z * x


def _full(arr):
    return pl.BlockSpec(arr.shape, lambda i: (0,) * arr.ndim)


def kernel(real_nodes, batch, vn_init, router_w1, router_b1, router_g,
           router_beta, router_w2, router_b2, r2v_w, r2v_b, r2v_g, r2v_beta,
           attn_in_w, attn_in_b, attn_out_w, attn_out_b, v2v_g, v2v_beta,
           v2r_w, v2r_b, v2r_g, v2r_beta, gru_w_ih, gru_w_hh, gru_b_ih,
           gru_b_hh):
    n = real_nodes.shape[0]
    assert n % BLK == 0
    nb = n // BLK
    row2 = lambda a: a.reshape(1, -1)
    batch2 = batch.astype(jnp.int32).reshape(n, 1)

    a_ins = [real_nodes, batch2, router_w1, row2(router_b1), row2(router_g),
             row2(router_beta), router_w2, row2(router_b2), r2v_w,
             row2(r2v_b), row2(r2v_g), row2(r2v_beta)]
    a_specs = ([pl.BlockSpec((BLK, H), lambda i: (i, 0)),
                pl.BlockSpec((BLK, 1), lambda i: (i, 0))]
               + [_full(a) for a in a_ins[2:]])
    agg, wts = pl.pallas_call(
        _phase_a,
        grid=(nb,),
        in_specs=a_specs,
        out_specs=[pl.BlockSpec((GV, H), lambda i: (0, 0)),
                   pl.BlockSpec((BLK, V), lambda i: (i, 0))],
        out_shape=[jax.ShapeDtypeStruct((GV, H), jnp.float32),
                   jax.ShapeDtypeStruct((n, V), jnp.float32)],
    )(*a_ins)

    vn_tile = jnp.tile(vn_init.reshape(V, H), (G, 1))
    b_ins = [agg, vn_tile, attn_in_w, row2(attn_in_b), attn_out_w,
             row2(attn_out_b), row2(v2v_g), row2(v2v_beta), v2r_w,
             row2(v2r_b), row2(v2r_g), row2(v2r_beta)]
    m2r = pl.pallas_call(
        _phase_b,
        out_shape=jax.ShapeDtypeStruct((GV, H), jnp.float32),
    )(*b_ins)

    c_ins = [real_nodes, batch2, wts, m2r, gru_w_ih, gru_w_hh,
             row2(gru_b_ih), row2(gru_b_hh)]
    c_specs = [pl.BlockSpec((BLK, H), lambda i: (i, 0)),
               pl.BlockSpec((BLK, 1), lambda i: (i, 0)),
               pl.BlockSpec((BLK, V), lambda i: (i, 0))] \
        + [_full(a) for a in c_ins[3:]]
    out = pl.pallas_call(
        _phase_c,
        grid=(nb,),
        in_specs=c_specs,
        out_specs=pl.BlockSpec((BLK, H), lambda i: (i, 0)),
        out_shape=jax.ShapeDtypeStruct((n, H), jnp.float32),
    )(*c_ins)
    return out
